# baseline (speedup 1.0000x reference)
import functools

import jax
import jax.numpy as jnp
import numpy as np
from jax import lax
from jax.experimental import layout
from jax.experimental import pallas as pl
from jax.experimental.pallas import tpu as pltpu
from jax.experimental.pallas import tpu_sc as plsc

ENTITIES = 100000
RELATIONS = 100000
DIM = 64
B = 4096
L = 200
MAX_INPUTS_LENGTH = 200

NC = 2
NS = 16
NW = NC * NS
LANES = 16

FLAT = B * L
PER_W = FLAT // NW
SEQ_PER_CHUNK = 2
CHUNK = SEQ_PER_CHUNK * L
NCHUNK = PER_W // CHUNK


def _fourier_pe(max_len, dim):
    input_positions = np.arange(max_len).reshape((-1, 1))
    embedding_positions = np.arange(dim).reshape((1, -1))
    relative = 2.0 * (embedding_positions // 2) / dim
    angles = input_positions / np.power(10000, relative)
    pe = np.zeros(angles.shape)
    pe[:, 0::2] = np.sin(angles[:, 0::2])
    pe[:, 1::2] = np.cos(angles[:, 1::2])
    return pe.astype(np.float32)


_PE_TILED = np.tile(_fourier_pe(MAX_INPUTS_LENGTH, DIM), (SEQ_PER_CHUNK, 1))


def _body(merged_hbm, ids_hbm, types_hbm, pe_hbm, out_hbm,
          pe_v, idx_v, types_v, rows0, rows1, sg0, sg1, so0, so1):
    wid = lax.axis_index("s") * NC + lax.axis_index("c")
    base_w = wid * PER_W
    rows = (rows0, rows1)
    sg = (sg0, sg1)
    so = (so0, so1)

    pltpu.sync_copy(pe_hbm, pe_v)
    pltpu.sync_copy(ids_hbm.at[pl.ds(base_w, PER_W)], idx_v)
    pltpu.sync_copy(types_hbm.at[pl.ds(base_w, PER_W)], types_v)

    def idx_body(i, c):
        s = pl.ds(i * LANES, LANES)
        idx_v[s] = idx_v[s] + types_v[s] * ENTITIES
        return c
    lax.fori_loop(0, PER_W // LANES, idx_body, 0, unroll=4)

    def gather_start(g, b):
        pltpu.async_copy(
            merged_hbm.at[idx_v.at[pl.ds(g * CHUNK, CHUNK)]], rows[b], sg[b])

    def gather_wait(g, b):
        pltpu.make_async_copy(
            merged_hbm.at[idx_v.at[pl.ds(g * CHUNK, CHUNK)]], rows[b], sg[b]
        ).wait()

    def write_start(g, b):
        pltpu.async_copy(
            rows[b], out_hbm.at[pl.ds(base_w + g * CHUNK, CHUNK)], so[b])

    def write_wait(g, b):
        pltpu.make_async_copy(
            rows[b], out_hbm.at[pl.ds(base_w + g * CHUNK, CHUNK)], so[b]
        ).wait()

    gather_start(0, 0)

    def outer(g2, c):
        for bi in range(2):
            g = g2 * 2 + bi
            gather_wait(g, bi)
            @pl.when(g >= 1)
            def _():
                write_wait(g - 1, 1 - bi)

            @pl.when(g + 1 < NCHUNK)
            def _():
                gather_start(g + 1, 1 - bi)

            def add_body(r, cc):
                for k in range(DIM // LANES):
                    s = pl.ds(k * LANES, LANES)
                    plsc.addupdate(rows[bi].at[r, s], pe_v[r, s])
                return cc
            lax.fori_loop(0, CHUNK, add_body, 0, unroll=4)

            write_start(g, bi)
        return c

    lax.fori_loop(0, NCHUNK // 2, outer, 0)
    write_wait(NCHUNK - 1, 1)


def _impl(object_ids, object_types, entity_embeddings, relation_embeddings):
    merged = jnp.concatenate([entity_embeddings, relation_embeddings], axis=0)
    ids = object_ids.reshape(-1).astype(jnp.int32)
    types = object_types.reshape(-1).astype(jnp.int32)
    pe = jnp.asarray(_PE_TILED)

    mesh = plsc.VectorSubcoreMesh(core_axis_name="c", subcore_axis_name="s")
    run = pl.kernel(
        _body,
        out_type=jax.ShapeDtypeStruct((FLAT, DIM), jnp.float32),
        mesh=mesh,
        scratch_types=[
            pltpu.VMEM((CHUNK, DIM), jnp.float32),
            pltpu.VMEM((PER_W,), jnp.int32),
            pltpu.VMEM((PER_W,), jnp.int32),
            pltpu.VMEM((CHUNK, DIM), jnp.float32),
            pltpu.VMEM((CHUNK, DIM), jnp.float32),
            pltpu.SemaphoreType.DMA,
            pltpu.SemaphoreType.DMA,
            pltpu.SemaphoreType.DMA,
            pltpu.SemaphoreType.DMA,
        ],
        compiler_params=pltpu.CompilerParams(use_tc_tiling_on_sc=False),
    )
    out = run(merged, ids, types, pe)
    res = out.reshape(B, L, DIM)
    return layout.with_layout_constraint(
        res, layout.Layout(major_to_minor=(0, 1, 2)))


_JITTED = None


def kernel(object_ids, object_types, entity_embeddings, relation_embeddings):
    global _JITTED
    if isinstance(object_ids, jax.core.Tracer):
        return _impl(object_ids, object_types, entity_embeddings,
                     relation_embeddings)
    if _JITTED is None:
        if hasattr(object_ids, "devices"):
            dev = next(iter(object_ids.devices()))
        else:
            dev = jax.devices()[0]
        sd = jax.sharding.SingleDeviceSharding(dev)
        row2d = layout.Format(layout.Layout(major_to_minor=(0, 1)), sd)
        out3d = layout.Format(
            layout.Layout(major_to_minor=(0, 1, 2), tiling=((8,),)), sd)
        _impl.__name__ = "kernel"
        _JITTED = jax.jit(_impl, in_shardings=(row2d, row2d, row2d, row2d),
                          out_shardings=out3d)
    return _JITTED(object_ids, object_types, entity_embeddings,
                   relation_embeddings)
